# baseline (speedup 1.0000x reference)
import jax
import jax.numpy as jnp
from jax.experimental import pallas as pl
from jax.experimental.pallas import tpu as pltpu

IN_F = 10
OUT_F = 5
CB = 262144


def _round_up(n: int, m: int) -> int:
    return ((n + m - 1) // m) * m


def _linear_t_kernel(xt_ref, w_ref, b_ref, o_ref):
    acc = jnp.dot(w_ref[...], xt_ref[...], preferred_element_type=jnp.float32)
    o_ref[...] = (acc + b_ref[...]).astype(o_ref.dtype)


@jax.jit
def _forward(x, w_packed, b_packed):
    B, in_f = x.shape
    assert in_f == IN_F

    w = w_packed[:, :OUT_F].T
    b = b_packed[:, :OUT_F].T

    b_pad = _round_up(B, CB)
    xp = jnp.pad(x, ((0, b_pad - B), (0, 0))) if b_pad != B else x
    xt = xp.T

    yt = pl.pallas_call(
        _linear_t_kernel,
        out_shape=jax.ShapeDtypeStruct((OUT_F, b_pad), x.dtype),
        grid=(b_pad // CB,),
        in_specs=[
            pl.BlockSpec((IN_F, CB), lambda i: (0, i)),
            pl.BlockSpec((OUT_F, IN_F), lambda i: (0, 0)),
            pl.BlockSpec((OUT_F, 1), lambda i: (0, 0)),
        ],
        out_specs=pl.BlockSpec((OUT_F, CB), lambda i: (0, i)),
        compiler_params=pltpu.CompilerParams(
            dimension_semantics=("parallel",),
        ),
    )(xt, w, b)

    y = yt.T
    return y[:B] if b_pad != B else y


def kernel(x, w_packed, b_packed):
    return _forward(x, w_packed, b_packed)


# speedup vs baseline: 1.0114x; 1.0114x over previous
import jax
import jax.numpy as jnp
from jax.experimental import pallas as pl
from jax.experimental.pallas import tpu as pltpu

IN_F = 10
OUT_F = 5
CB = 131072


def _round_up(n: int, m: int) -> int:
    return ((n + m - 1) // m) * m


def _linear_t_kernel(xt_ref, w_ref, b_ref, o_ref):
    acc = jnp.dot(w_ref[...], xt_ref[...], preferred_element_type=jnp.float32)
    o_ref[...] = (acc + b_ref[...]).astype(o_ref.dtype)


@jax.jit
def _forward(x, w_packed, b_packed):
    B, in_f = x.shape
    assert in_f == IN_F

    w = w_packed[:, :OUT_F].T
    b = b_packed[:, :OUT_F].T

    b_pad = _round_up(B, CB)
    xp = jnp.pad(x, ((0, b_pad - B), (0, 0))) if b_pad != B else x
    xt = xp.T

    yt = pl.pallas_call(
        _linear_t_kernel,
        out_shape=jax.ShapeDtypeStruct((OUT_F, b_pad), x.dtype),
        grid=(b_pad // CB,),
        in_specs=[
            pl.BlockSpec((IN_F, CB), lambda i: (0, i)),
            pl.BlockSpec((OUT_F, IN_F), lambda i: (0, 0)),
            pl.BlockSpec((OUT_F, 1), lambda i: (0, 0)),
        ],
        out_specs=pl.BlockSpec((OUT_F, CB), lambda i: (0, i)),
        compiler_params=pltpu.CompilerParams(
            dimension_semantics=("parallel",),
        ),
    )(xt, w, b)

    y = yt.T
    return y[:B] if b_pad != B else y


def kernel(x, w_packed, b_packed):
    return _forward(x, w_packed, b_packed)
